# baseline (speedup 1.0000x reference)
import numpy as np
import jax
import jax.numpy as jnp
from jax.experimental import pallas as pl

L_SLICES = [(0, 1), (1, 4), (4, 9), (9, 16)]
NUM_POINTS = [1024, 256, 64, 16]
RADIUS = [0.2, 0.4, 0.8]
PATCH = 32
NUM_SHELLS = 3
GSCALE = 0.69314718056 * 9
BIG = 1e30


def _shb_np():
    n = 64
    idx = np.arange(0, n, dtype=np.float64) + 0.5
    phi = np.arccos(1 - 2 * idx / n)
    theta = np.pi * (1 + 5 ** 0.5) * idx
    x = (np.cos(theta) * np.sin(phi)).astype(np.float32)
    y = (np.sin(theta) * np.sin(phi)).astype(np.float32)
    z = np.cos(phi).astype(np.float32)
    one = np.ones_like(x)
    comps = [one,
             y, z, x,
             x * y, y * z, 3 * z * z - 1, z * x, x * x - y * y,
             y * (3 * x * x - y * y), x * y * z, y * (5 * z * z - 1),
             z * (5 * z * z - 3),
             x * (5 * z * z - 1), z * (x * x - y * y), x * (x * x - 3 * y * y)]
    return np.stack(comps, axis=-1).astype(np.float32)


def _sh_cols(ux, uy, uz):
    one = jnp.ones_like(ux)
    return [one,
            uy, uz, ux,
            ux * uy, uy * uz, 3 * uz * uz - 1, uz * ux, ux * ux - uy * uy,
            uy * (3 * ux * ux - uy * uy), ux * uy * uz,
            uy * (5 * uz * uz - 1), uz * (5 * uz * uz - 3),
            ux * (5 * uz * uz - 1), uz * (ux * ux - uy * uy),
            ux * (ux * ux - 3 * uy * uy)]


def _sort_body(kc_ref, kr_ref, x_ref, o_ref):
    kc = kc_ref[0]
    kr = kr_ref[0]
    xv = x_ref[0]
    n = kc.shape[0]
    less = (kr < kc).astype(jnp.float32)
    eq = kr == kc
    ii = jax.lax.broadcasted_iota(jnp.int32, (n, n), 0)
    jj = jax.lax.broadcasted_iota(jnp.int32, (n, n), 1)
    eqlt = jnp.where(jnp.logical_and(eq, jj < ii), 1.0, 0.0)
    rank = jnp.sum(less + eqlt, axis=1, keepdims=True)
    rr = jax.lax.broadcasted_iota(jnp.int32, (n, n), 1)
    perm = (rank.astype(jnp.int32) == rr).astype(jnp.float32)
    srt = jax.lax.dot_general(perm, xv, (((0,), (0,)), ((), ())),
                              preferred_element_type=jnp.float32)
    o_ref[0] = srt


def _dot(a, b):
    return jnp.dot(a, b, preferred_element_type=jnp.float32)


def _knn_geom(T, S, Y, Ns, Nt, radius, accum_init, accum_fn):
    TS = jax.lax.dot_general(T, S, (((1,), (1,)), ((), ())),
                             preferred_element_type=jnp.float32)
    t2 = jnp.sum(T * T, axis=1, keepdims=True)
    ones_13 = jnp.zeros((1, 3), jnp.float32) + 1.0
    s2 = jax.lax.dot_general(ones_13, S * S,
                             (((1,), (1,)), ((), ())),
                             preferred_element_type=jnp.float32)
    d2 = t2 - 2.0 * TS + s2
    iota_j = jax.lax.broadcasted_iota(jnp.int32, (Nt, Ns), 1)
    centers = 0.5 * jax.lax.broadcasted_iota(
        jnp.int32, (1, 3), 1).astype(jnp.float32)

    def step(_, carry):
        d2m = carry[0]
        acc = carry[1:]
        mn = jnp.min(d2m, axis=1, keepdims=True)
        cand = jnp.where(d2m == mn, iota_j, Ns)
        idx = jnp.min(cand, axis=1, keepdims=True)
        ohb = iota_j == idx
        oh = ohb.astype(jnp.float32)
        Gk = _dot(oh, Y)
        rel = Gk[:, 0:3] - T
        dist = jnp.sqrt(jnp.sum(rel * rel, axis=1, keepdims=True) + 1e-8)
        u = rel / dist
        sh = jnp.concatenate(_sh_cols(u[:, 0:1], u[:, 1:2], u[:, 2:3]),
                             axis=1)
        dn = dist / radius
        g = jnp.exp(-GSCALE * (dn - centers) ** 2)
        g = g * (dn <= 1.0).astype(jnp.float32)
        kern = sh[:, :, None] * g[:, None, :]
        new_acc = accum_fn(acc, Gk, kern, g)
        d2m = jnp.where(ohb, BIG, d2m)
        return (d2m,) + tuple(new_acc)

    carry = jax.lax.fori_loop(0, PATCH, step, (d2,) + tuple(accum_init))
    return carry[1:]


def _tail(z_list, Nt, u, shb, mw1, mb1, mw2, mb2):
    cat = jnp.concatenate(z_list, axis=1)
    ct = jnp.transpose(cat, (1, 0, 2))
    sig = jax.lax.dot_general(shb, ct, (((1,), (0,)), ((), ())),
                              preferred_element_type=jnp.float32)
    sig = jnp.maximum(sig, 0.0)
    X = sig.reshape(64 * Nt, u)
    X = jnp.maximum(_dot(X, mw1) + mb1, 0.0)
    X = jnp.maximum(_dot(X, mw2) + mb2, 0.0)
    return X


def _make_scale0(Ns, Nt, radius, u, c2):
    def body(T_ref, Y_ref, shb_ref, shbT_ref,
             w0_ref, w1_ref, w2_ref, w3_ref, b0_ref,
             mw1_ref, mb1_ref, mw2_ref, mb2_ref, o_ref):
        T = T_ref[0]
        Y = Y_ref[0]

        tA0 = jnp.zeros((Nt, 16, 3), jnp.float32)
        tB1 = jnp.zeros((Nt, 3, 3), jnp.float32)

        def accum(acc, Gk, kern, g):
            tA, tB = acc
            tA = tA + kern
            yzx_s = jnp.concatenate(
                [Gk[:, 1:2], Gk[:, 2:3], Gk[:, 0:1]], axis=1)
            tB = tB + yzx_s[:, :, None] * kern[:, 0, :][:, None, :]
            return (tA, tB)

        tA, tB1 = _knn_geom(T, S=Y, Y=Y, Ns=Ns, Nt=Nt, radius=radius,
                            accum_init=(tA0, tB1), accum_fn=accum)
        inv = jnp.float32(1.0 / PATCH)
        yzx_t = jnp.concatenate([T[:, 1:2], T[:, 2:3], T[:, 0:1]],
                                axis=1)[:, :, None]
        out0 = jnp.concatenate([tA[:, 0:1, :], tA[:, 0:1, :]], -1) * inv
        out1 = jnp.concatenate(
            [tA[:, 1:4, :] * inv, tB1 * inv, yzx_t], -1)
        out2 = tA[:, 4:9, :] * inv
        out3 = tA[:, 9:16, :] * inv
        ws = [w0_ref[...], w1_ref[...], w2_ref[...], w3_ref[...]]
        dims = [(1, 6), (3, 7), (5, 3), (7, 3)]
        z_list = []
        for l, (o_l, (m, din)) in enumerate(zip([out0, out1, out2, out3],
                                                dims)):
            z = _dot(o_l.reshape(Nt * m, din), ws[l])
            if l == 0:
                z = z + b0_ref[...]
            z_list.append(z.reshape(Nt, m, u))
        X = _tail(z_list, Nt, u, shb_ref[...],
                  mw1_ref[...], mb1_ref[...], mw2_ref[...], mb2_ref[...])
        X3 = X.reshape(64, Nt, c2)
        YS = jax.lax.dot_general(shbT_ref[...], X3, (((1,), (0,)), ((), ())),
                                 preferred_element_type=jnp.float32)
        o_ref[0] = YS
    return body


def _make_scale(Ns, Nt, radius, c_in, u, c2, final):
    c0 = c_in
    cls = [c_in, c_in + 1, c_in, c_in]
    offs = []
    off = 3 + c0
    for l in range(4):
        m = 2 * l + 1
        if l == 0:
            offs.append(3)
        else:
            offs.append(off)
            off += m * cls[l]

    def body(*refs):
        (T_ref, Y_ref, shb_ref, shbT_ref,
         w0_ref, w1_ref, w2_ref, w3_ref, b0_ref,
         mw1_ref, mb1_ref, mw2_ref, mb2_ref) = refs[:13]
        if final:
            hw1_ref, hb1_ref, hw2_ref, hb2_ref, o_ref = refs[13:]
        else:
            o_ref = refs[13]
        T = T_ref[0]
        Y = Y_ref[0]
        S = Y[:, 0:3]

        tA0 = jnp.zeros((Nt, 16, 3, c0), jnp.float32)
        tBs0 = tuple(jnp.zeros((Nt, 2 * l + 1, 3, cls[l]), jnp.float32)
                     for l in range(4))

        def accum(acc, Gk, kern, g):
            tA = acc[0]
            tBs = list(acc[1:])
            g0 = Gk[:, 3:3 + c0]
            tA = tA + kern[:, :, :, None] * g0[:, None, None, :]
            k0 = kern[:, 0, :]
            for l in range(4):
                m = 2 * l + 1
                gl = Gk[:, offs[l]:offs[l] + m * cls[l]].reshape(Nt, m,
                                                                 cls[l])
                tBs[l] = tBs[l] + k0[:, None, :, None] * gl[:, :, None, :]
            return (tA,) + tuple(tBs)

        acc = _knn_geom(T, S=S, Y=Y, Ns=Ns, Nt=Nt, radius=radius,
                        accum_init=(tA0,) + tBs0, accum_fn=accum)
        tA = acc[0]
        tBs = acc[1:]
        inv = jnp.float32(1.0 / PATCH)
        yzx_t = jnp.concatenate([T[:, 1:2], T[:, 2:3], T[:, 0:1]],
                                axis=1)[:, :, None]
        ws = [w0_ref[...], w1_ref[...], w2_ref[...], w3_ref[...]]
        z_list = []
        for l in range(4):
            a, b = L_SLICES[l]
            m = 2 * l + 1
            parts = [tA[:, a:b].reshape(Nt, m, 3 * c0) * inv,
                     tBs[l].reshape(Nt, m, 3 * cls[l]) * inv]
            if l == 1:
                parts.append(yzx_t)
            o_l = jnp.concatenate(parts, axis=-1)
            din = o_l.shape[-1]
            z = _dot(o_l.reshape(Nt * m, din), ws[l])
            if l == 0:
                z = z + b0_ref[...]
            z_list.append(z.reshape(Nt, m, u))
        X = _tail(z_list, Nt, u, shb_ref[...],
                  mw1_ref[...], mb1_ref[...], mw2_ref[...], mb2_ref[...])
        X3 = X.reshape(64, Nt, c2)
        if final:
            gmax = jnp.max(X3, axis=1)
            h = jnp.maximum(_dot(gmax, hw1_ref[...]) + hb1_ref[...], 0.0)
            code = _dot(h, hw2_ref[...]) + hb2_ref[...]
            lat = _dot(shbT_ref[...], code) * jnp.float32(1.0 / 64.0)
            o_ref[0] = lat
        else:
            YS = jax.lax.dot_general(shbT_ref[...], X3,
                                     (((1,), (0,)), ((), ())),
                                     preferred_element_type=jnp.float32)
            o_ref[0] = YS
    return body


def _bspec(shape, batched):
    if batched:
        blk = (1,) + shape[1:]
        nd = len(shape) - 1
        return pl.BlockSpec(blk, lambda b, _nd=nd: (b,) + (0,) * _nd)
    return pl.BlockSpec(shape, lambda b, _nd=len(shape): (0,) * _nd)


def _call(body, outs_shape, args_batched):
    B = outs_shape[0]
    in_specs = [_bspec(a.shape, bt) for a, bt in args_batched]
    out_spec = _bspec(outs_shape, True)
    return pl.pallas_call(
        body,
        grid=(B,),
        in_specs=in_specs,
        out_specs=out_spec,
        out_shape=jax.ShapeDtypeStruct(outs_shape, jnp.float32),
    )(*[a for a, _ in args_batched])


def kernel(x, params):
    B = x.shape[0]
    p = params
    shb = jnp.asarray(_shb_np())
    mscale = np.zeros((16,), np.float32)
    for l in range(4):
        a, b = L_SLICES[l]
        mscale[a:b] = (2 * l + 1) / 64.0
    shbT_s = jnp.asarray(_shb_np().T * mscale[:, None])
    shbT_f = jnp.asarray(_shb_np().T)

    kc = x[..., 0:1]
    kr = x[..., 0][:, None, :]
    srt = pl.pallas_call(
        _sort_body,
        grid=(B,),
        in_specs=[_bspec(kc.shape, True), _bspec(kr.shape, True),
                  _bspec(x.shape, True)],
        out_specs=_bspec(x.shape, True),
        out_shape=jax.ShapeDtypeStruct(x.shape, jnp.float32),
    )(kc, kr, x)

    points = [srt]
    for i in range(3):
        pts = points[-1]
        points.append(pts.reshape(B, NUM_POINTS[i + 1],
                                  NUM_POINTS[i] // NUM_POINTS[i + 1],
                                  3)[:, :, 0, :])

    def mlp_args(i):
        out = []
        for j in range(2):
            out.append((p['mlp%d_%d_W' % (i, j)], False))
            out.append((p['mlp%d_%d_b' % (i, j)][None, :], False))
        return out

    def eq_args(i):
        out = [(p['eq%d_%d' % (i, l)], False) for l in range(4)]
        out.append((p['eqb%d' % i][None, :], False))
        return out

    Nt0 = NUM_POINTS[1]
    body0 = _make_scale0(NUM_POINTS[0], Nt0, RADIUS[0], u=32, c2=32)
    args0 = ([(points[1], True), (points[0], True),
              (shb, False), (shbT_s, False)]
             + eq_args(0) + mlp_args(0))
    YS0 = _call(body0, (B, 16, Nt0, 32), args0)

    def build_Y(pts, YS, c):
        Nt = pts.shape[1]
        parts = [pts]
        for l in range(4):
            a, b = L_SLICES[l]
            yl = jnp.transpose(YS[:, a:b], (0, 2, 1, 3))
            if l == 1:
                yzx = jnp.stack([pts[..., 1], pts[..., 2], pts[..., 0]],
                                axis=-1)[..., None]
                yl = jnp.concatenate([yl, yzx], axis=-1)
            parts.append(yl.reshape(B, Nt, -1))
        return jnp.concatenate(parts, axis=-1)

    Y1 = build_Y(points[1], YS0, 32)
    Nt1 = NUM_POINTS[2]
    body1 = _make_scale(NUM_POINTS[1], Nt1, RADIUS[1], c_in=32, u=64,
                        c2=64, final=False)
    args1 = ([(points[2], True), (Y1, True), (shb, False), (shbT_s, False)]
             + eq_args(1) + mlp_args(1))
    YS1 = _call(body1, (B, 16, Nt1, 64), args1)

    Y2 = build_Y(points[2], YS1, 64)
    Nt2 = NUM_POINTS[3]
    body2 = _make_scale(NUM_POINTS[2], Nt2, RADIUS[2], c_in=64, u=128,
                        c2=256, final=True)
    args2 = ([(points[3], True), (Y2, True), (shb, False), (shbT_f, False)]
             + eq_args(2) + mlp_args(2)
             + [(p['code_mlp_W'], False), (p['code_mlp_b'][None, :], False),
                (p['code_W'], False), (p['code_b'][None, :], False)])
    latent = _call(body2, (B, 16, 128), args2)
    return latent


# speedup vs baseline: 118.1296x; 118.1296x over previous
import numpy as np
import jax
import jax.numpy as jnp
from jax.experimental import pallas as pl

L_SLICES = [(0, 1), (1, 4), (4, 9), (9, 16)]
NUM_POINTS = [1024, 256, 64, 16]
RADIUS = [0.2, 0.4, 0.8]
PATCH = 32
NUM_SHELLS = 3
GSCALE = 0.69314718056 * 9
BIG = 1e30


def _shb_np():
    n = 64
    idx = np.arange(0, n, dtype=np.float64) + 0.5
    phi = np.arccos(1 - 2 * idx / n)
    theta = np.pi * (1 + 5 ** 0.5) * idx
    x = (np.cos(theta) * np.sin(phi)).astype(np.float32)
    y = (np.sin(theta) * np.sin(phi)).astype(np.float32)
    z = np.cos(phi).astype(np.float32)
    one = np.ones_like(x)
    comps = [one,
             y, z, x,
             x * y, y * z, 3 * z * z - 1, z * x, x * x - y * y,
             y * (3 * x * x - y * y), x * y * z, y * (5 * z * z - 1),
             z * (5 * z * z - 3),
             x * (5 * z * z - 1), z * (x * x - y * y), x * (x * x - 3 * y * y)]
    return np.stack(comps, axis=-1).astype(np.float32)


def _sh_cols(ux, uy, uz):
    one = jnp.ones_like(ux)
    return [one,
            uy, uz, ux,
            ux * uy, uy * uz, 3 * uz * uz - 1, uz * ux, ux * ux - uy * uy,
            uy * (3 * ux * ux - uy * uy), ux * uy * uz,
            uy * (5 * uz * uz - 1), uz * (5 * uz * uz - 3),
            ux * (5 * uz * uz - 1), uz * (ux * ux - uy * uy),
            ux * (ux * ux - 3 * uy * uy)]


def _sort_body(kc_ref, kr_ref, x_ref, o_ref):
    kc = kc_ref[0]
    kr = kr_ref[0]
    xv = x_ref[0]
    n = kc.shape[0]
    less = (kr < kc).astype(jnp.float32)
    eq = kr == kc
    ii = jax.lax.broadcasted_iota(jnp.int32, (n, n), 0)
    jj = jax.lax.broadcasted_iota(jnp.int32, (n, n), 1)
    eqlt = jnp.where(jnp.logical_and(eq, jj < ii), 1.0, 0.0)
    rank = jnp.sum(less + eqlt, axis=1, keepdims=True)
    rr = jax.lax.broadcasted_iota(jnp.int32, (n, n), 1)
    perm = (rank.astype(jnp.int32) == rr).astype(jnp.float32)
    srt = jax.lax.dot_general(perm, xv, (((0,), (0,)), ((), ())),
                              preferred_element_type=jnp.float32)
    o_ref[0] = srt


def _dot(a, b):
    return jnp.dot(a, b, preferred_element_type=jnp.float32)


def _knn_geom(T, Yg, Ns, Nt, radius, accum_init, accum_fn):
    S = Yg[:, 0:3]
    TS = jax.lax.dot_general(T, S, (((1,), (1,)), ((), ())),
                             preferred_element_type=jnp.float32)
    t2 = jnp.sum(T * T, axis=1, keepdims=True)
    ones_13 = jnp.zeros((1, 3), jnp.float32) + 1.0
    s2 = jax.lax.dot_general(ones_13, S * S,
                             (((1,), (1,)), ((), ())),
                             preferred_element_type=jnp.float32)
    d2 = jnp.clip(t2 - 2.0 * TS + s2, 0.0, 1.99)
    iota_j = jax.lax.broadcasted_iota(jnp.int32, (Nt, Ns), 1)
    key0 = (d2 * 1048576.0).astype(jnp.int32) * 1024 + iota_j
    centers = 0.5 * jax.lax.broadcasted_iota(
        jnp.int32, (1, 3), 1).astype(jnp.float32)
    intmax = jnp.int32(2147483647)

    def step(_, carry):
        key = carry[0]
        acc = carry[1:]
        km = jnp.min(key, axis=1, keepdims=True)
        ohb = key == km
        oh = ohb.astype(jnp.float32)
        Gk = _dot(oh, Yg)
        rel = Gk[:, 0:3] - T
        dist = jnp.sqrt(jnp.sum(rel * rel, axis=1, keepdims=True) + 1e-8)
        u = rel / dist
        sh = jnp.concatenate(_sh_cols(u[:, 0:1], u[:, 1:2], u[:, 2:3]),
                             axis=1)
        dn = dist / radius
        g = jnp.exp(-GSCALE * (dn - centers) ** 2)
        g = g * (dn <= 1.0).astype(jnp.float32)
        new_acc = accum_fn(acc, oh, Gk, sh, g)
        key = jnp.where(ohb, intmax, key)
        return (key,) + tuple(new_acc)

    carry = jax.lax.fori_loop(0, PATCH, step, (key0,) + tuple(accum_init))
    return carry[1:]


def _tail(z_list, Nt, u, shb, mw1, mb1, mw2, mb2):
    cat = jnp.concatenate(z_list, axis=1)
    ct = jnp.transpose(cat, (1, 0, 2))
    sig = jax.lax.dot_general(shb, ct, (((1,), (0,)), ((), ())),
                              preferred_element_type=jnp.float32)
    sig = jnp.maximum(sig, 0.0)
    X = sig.reshape(64 * Nt, u)
    X = jnp.maximum(_dot(X, mw1) + mb1, 0.0)
    X = jnp.maximum(_dot(X, mw2) + mb2, 0.0)
    return X


def _make_scale0(Ns, Nt, radius, u, c2):
    def body(T_ref, Y_ref, shb_ref, shbT_ref,
             w0_ref, w1_ref, w2_ref, w3_ref, b0_ref,
             mw1_ref, mb1_ref, mw2_ref, mb2_ref, o_ref):
        T = T_ref[0]
        Y = Y_ref[0]

        init = tuple(jnp.zeros((Nt, 16), jnp.float32) for _ in range(3)) \
            + tuple(jnp.zeros((Nt, 3), jnp.float32) for _ in range(3))

        def accum(acc, oh, Gk, sh, g):
            yzx_s = jnp.concatenate(
                [Gk[:, 1:2], Gk[:, 2:3], Gk[:, 0:1]], axis=1)
            out = []
            for s in range(3):
                out.append(acc[s] + sh * g[:, s:s + 1])
            for s in range(3):
                out.append(acc[3 + s] + yzx_s * g[:, s:s + 1])
            return tuple(out)

        acc = _knn_geom(T, Y, Ns=Ns, Nt=Nt, radius=radius,
                        accum_init=init, accum_fn=accum)
        tAs = acc[0:3]
        tB1s = acc[3:6]
        inv = jnp.float32(1.0 / PATCH)
        yzx_t = jnp.concatenate([T[:, 1:2], T[:, 2:3], T[:, 0:1]],
                                axis=1)[:, :, None]

        def apart(a, b):
            return jnp.concatenate(
                [tAs[s][:, a:b, None] for s in range(3)], axis=-1)

        t0 = apart(0, 1)
        out0 = jnp.concatenate([t0, t0], -1) * inv
        tB1 = jnp.concatenate([tB1s[s][:, :, None] for s in range(3)], -1)
        out1 = jnp.concatenate(
            [apart(1, 4) * inv, tB1 * inv, yzx_t], -1)
        out2 = apart(4, 9) * inv
        out3 = apart(9, 16) * inv
        ws = [w0_ref[...], w1_ref[...], w2_ref[...], w3_ref[...]]
        dims = [(1, 6), (3, 7), (5, 3), (7, 3)]
        z_list = []
        for l, (o_l, (m, din)) in enumerate(zip([out0, out1, out2, out3],
                                                dims)):
            z = _dot(o_l.reshape(Nt * m, din), ws[l])
            if l == 0:
                z = z + b0_ref[...]
            z_list.append(z.reshape(Nt, m, u))
        X = _tail(z_list, Nt, u, shb_ref[...],
                  mw1_ref[...], mb1_ref[...], mw2_ref[...], mb2_ref[...])
        X3 = X.reshape(64, Nt, c2)
        YS = jax.lax.dot_general(shbT_ref[...], X3, (((1,), (0,)), ((), ())),
                                 preferred_element_type=jnp.float32)
        o_ref[0] = YS
    return body


def _make_scale(Ns, Nt, radius, c_in, u, c2, final):
    c0 = c_in
    cls = [c_in, c_in + 1, c_in, c_in]
    offs = []
    off = 3 + c0
    for l in range(4):
        m = 2 * l + 1
        if l == 0:
            offs.append(3)
        else:
            offs.append(off)
            off += m * cls[l]

    def body(*refs):
        (T_ref, Y_ref, shb_ref, shbT_ref,
         w0_ref, w1_ref, w2_ref, w3_ref, b0_ref,
         mw1_ref, mb1_ref, mw2_ref, mb2_ref) = refs[:13]
        if final:
            hw1_ref, hb1_ref, hw2_ref, hb2_ref, o_ref = refs[13:]
        else:
            o_ref = refs[13]
        T = T_ref[0]
        Y = Y_ref[0]
        Yg = Y[:, 0:3 + c0]
        YL = Y[:, 3:]

        init = tuple(jnp.zeros((Nt, 16, c0), jnp.float32) for _ in range(3)) \
            + tuple(jnp.zeros((Nt, Ns), jnp.float32) for _ in range(3))

        def accum(acc, oh, Gk, sh, g):
            g0 = Gk[:, 3:3 + c0]
            out = []
            for s in range(3):
                g0s = g0 * g[:, s:s + 1]
                out.append(acc[s] + sh[:, :, None] * g0s[:, None, :])
            for s in range(3):
                out.append(acc[3 + s] + oh * g[:, s:s + 1])
            return tuple(out)

        acc = _knn_geom(T, Yg, Ns=Ns, Nt=Nt, radius=radius,
                        accum_init=init, accum_fn=accum)
        tAs = acc[0:3]
        ows = acc[3:6]
        tBall = [_dot(ow, YL) for ow in ows]
        inv = jnp.float32(1.0 / PATCH)
        yzx_t = jnp.concatenate([T[:, 1:2], T[:, 2:3], T[:, 0:1]],
                                axis=1)[:, :, None]
        ws = [w0_ref[...], w1_ref[...], w2_ref[...], w3_ref[...]]
        z_list = []
        for l in range(4):
            a, b = L_SLICES[l]
            m = 2 * l + 1
            c = cls[l]
            o = offs[l] - 3
            tA_l = jnp.concatenate([tAs[s][:, a:b, :] for s in range(3)],
                                   axis=-1)
            tB_l = jnp.concatenate(
                [tBall[s][:, o:o + m * c].reshape(Nt, m, c)
                 for s in range(3)], axis=-1)
            parts = [tA_l * inv, tB_l * inv]
            if l == 1:
                parts.append(yzx_t)
            o_l = jnp.concatenate(parts, axis=-1)
            din = o_l.shape[-1]
            z = _dot(o_l.reshape(Nt * m, din), ws[l])
            if l == 0:
                z = z + b0_ref[...]
            z_list.append(z.reshape(Nt, m, u))
        X = _tail(z_list, Nt, u, shb_ref[...],
                  mw1_ref[...], mb1_ref[...], mw2_ref[...], mb2_ref[...])
        X3 = X.reshape(64, Nt, c2)
        if final:
            gmax = jnp.max(X3, axis=1)
            h = jnp.maximum(_dot(gmax, hw1_ref[...]) + hb1_ref[...], 0.0)
            code = _dot(h, hw2_ref[...]) + hb2_ref[...]
            lat = _dot(shbT_ref[...], code) * jnp.float32(1.0 / 64.0)
            o_ref[0] = lat
        else:
            YS = jax.lax.dot_general(shbT_ref[...], X3,
                                     (((1,), (0,)), ((), ())),
                                     preferred_element_type=jnp.float32)
            o_ref[0] = YS
    return body


def _bspec(shape, batched):
    if batched:
        blk = (1,) + shape[1:]
        nd = len(shape) - 1
        return pl.BlockSpec(blk, lambda b, _nd=nd: (b,) + (0,) * _nd)
    return pl.BlockSpec(shape, lambda b, _nd=len(shape): (0,) * _nd)


def _call(body, outs_shape, args_batched):
    B = outs_shape[0]
    in_specs = [_bspec(a.shape, bt) for a, bt in args_batched]
    out_spec = _bspec(outs_shape, True)
    return pl.pallas_call(
        body,
        grid=(B,),
        in_specs=in_specs,
        out_specs=out_spec,
        out_shape=jax.ShapeDtypeStruct(outs_shape, jnp.float32),
    )(*[a for a, _ in args_batched])


def kernel(x, params):
    B = x.shape[0]
    p = params
    shb = jnp.asarray(_shb_np())
    mscale = np.zeros((16,), np.float32)
    for l in range(4):
        a, b = L_SLICES[l]
        mscale[a:b] = (2 * l + 1) / 64.0
    shbT_s = jnp.asarray(_shb_np().T * mscale[:, None])
    shbT_f = jnp.asarray(_shb_np().T)

    kc = x[..., 0:1]
    kr = x[..., 0][:, None, :]
    srt = pl.pallas_call(
        _sort_body,
        grid=(B,),
        in_specs=[_bspec(kc.shape, True), _bspec(kr.shape, True),
                  _bspec(x.shape, True)],
        out_specs=_bspec(x.shape, True),
        out_shape=jax.ShapeDtypeStruct(x.shape, jnp.float32),
    )(kc, kr, x)

    points = [srt]
    for i in range(3):
        pts = points[-1]
        points.append(pts.reshape(B, NUM_POINTS[i + 1],
                                  NUM_POINTS[i] // NUM_POINTS[i + 1],
                                  3)[:, :, 0, :])

    def mlp_args(i):
        out = []
        for j in range(2):
            out.append((p['mlp%d_%d_W' % (i, j)], False))
            out.append((p['mlp%d_%d_b' % (i, j)][None, :], False))
        return out

    def eq_args(i):
        out = [(p['eq%d_%d' % (i, l)], False) for l in range(4)]
        out.append((p['eqb%d' % i][None, :], False))
        return out

    Nt0 = NUM_POINTS[1]
    body0 = _make_scale0(NUM_POINTS[0], Nt0, RADIUS[0], u=32, c2=32)
    args0 = ([(points[1], True), (points[0], True),
              (shb, False), (shbT_s, False)]
             + eq_args(0) + mlp_args(0))
    YS0 = _call(body0, (B, 16, Nt0, 32), args0)

    def build_Y(pts, YS, c):
        Nt = pts.shape[1]
        parts = [pts]
        for l in range(4):
            a, b = L_SLICES[l]
            yl = jnp.transpose(YS[:, a:b], (0, 2, 1, 3))
            if l == 1:
                yzx = jnp.stack([pts[..., 1], pts[..., 2], pts[..., 0]],
                                axis=-1)[..., None]
                yl = jnp.concatenate([yl, yzx], axis=-1)
            parts.append(yl.reshape(B, Nt, -1))
        return jnp.concatenate(parts, axis=-1)

    Y1 = build_Y(points[1], YS0, 32)
    Nt1 = NUM_POINTS[2]
    body1 = _make_scale(NUM_POINTS[1], Nt1, RADIUS[1], c_in=32, u=64,
                        c2=64, final=False)
    args1 = ([(points[2], True), (Y1, True), (shb, False), (shbT_s, False)]
             + eq_args(1) + mlp_args(1))
    YS1 = _call(body1, (B, 16, Nt1, 64), args1)

    Y2 = build_Y(points[2], YS1, 64)
    Nt2 = NUM_POINTS[3]
    body2 = _make_scale(NUM_POINTS[2], Nt2, RADIUS[2], c_in=64, u=128,
                        c2=256, final=True)
    args2 = ([(points[3], True), (Y2, True), (shb, False), (shbT_f, False)]
             + eq_args(2) + mlp_args(2)
             + [(p['code_mlp_W'], False), (p['code_mlp_b'][None, :], False),
                (p['code_W'], False), (p['code_b'][None, :], False)])
    latent = _call(body2, (B, 16, 128), args2)
    return latent


# speedup vs baseline: 194.1000x; 1.6431x over previous
import numpy as np
import jax
import jax.numpy as jnp
from jax.experimental import pallas as pl

L_SLICES = [(0, 1), (1, 4), (4, 9), (9, 16)]
NUM_POINTS = [1024, 256, 64, 16]
RADIUS = [0.2, 0.4, 0.8]
PATCH = 32
NUM_SHELLS = 3
GSCALE = 0.69314718056 * 9
BIG = 1e30


def _shb_np():
    n = 64
    idx = np.arange(0, n, dtype=np.float64) + 0.5
    phi = np.arccos(1 - 2 * idx / n)
    theta = np.pi * (1 + 5 ** 0.5) * idx
    x = (np.cos(theta) * np.sin(phi)).astype(np.float32)
    y = (np.sin(theta) * np.sin(phi)).astype(np.float32)
    z = np.cos(phi).astype(np.float32)
    one = np.ones_like(x)
    comps = [one,
             y, z, x,
             x * y, y * z, 3 * z * z - 1, z * x, x * x - y * y,
             y * (3 * x * x - y * y), x * y * z, y * (5 * z * z - 1),
             z * (5 * z * z - 3),
             x * (5 * z * z - 1), z * (x * x - y * y), x * (x * x - 3 * y * y)]
    return np.stack(comps, axis=-1).astype(np.float32)


def _sh_cols(ux, uy, uz):
    one = jnp.ones_like(ux)
    return [one,
            uy, uz, ux,
            ux * uy, uy * uz, 3 * uz * uz - 1, uz * ux, ux * ux - uy * uy,
            uy * (3 * ux * ux - uy * uy), ux * uy * uz,
            uy * (5 * uz * uz - 1), uz * (5 * uz * uz - 3),
            ux * (5 * uz * uz - 1), uz * (ux * ux - uy * uy),
            ux * (ux * ux - 3 * uy * uy)]


def _sort_body(kc_ref, kr_ref, x_ref, o_ref):
    kc = kc_ref[0]
    kr = kr_ref[0]
    xv = x_ref[0]
    n = kc.shape[0]
    less = (kr < kc).astype(jnp.float32)
    eq = kr == kc
    ii = jax.lax.broadcasted_iota(jnp.int32, (n, n), 0)
    jj = jax.lax.broadcasted_iota(jnp.int32, (n, n), 1)
    eqlt = jnp.where(jnp.logical_and(eq, jj < ii), 1.0, 0.0)
    rank = jnp.sum(less + eqlt, axis=1, keepdims=True)
    rr = jax.lax.broadcasted_iota(jnp.int32, (n, n), 1)
    perm = (rank.astype(jnp.int32) == rr).astype(jnp.float32)
    srt = jax.lax.dot_general(perm, xv, (((0,), (0,)), ((), ())),
                              preferred_element_type=jnp.float32)
    o_ref[0] = srt


def _dot(a, b):
    return jnp.dot(a, b, preferred_element_type=jnp.float32)


def _knn_geom(TT, YgT, Ns, Nt, radius, accum_init, accum_fn):
    S_T = YgT[0:3, :]
    ST = jax.lax.dot_general(S_T, TT, (((0,), (0,)), ((), ())),
                             preferred_element_type=jnp.float32)
    s2col = jnp.transpose(jnp.sum(S_T * S_T, axis=0, keepdims=True))
    t2row = jnp.sum(TT * TT, axis=0, keepdims=True)
    d2 = jnp.clip(s2col - 2.0 * ST + t2row, 0.0, 1.99)
    iota_i = jax.lax.broadcasted_iota(jnp.int32, (Ns, Nt), 0)
    key0 = (d2 * 1048576.0).astype(jnp.int32) * 1024 + iota_i
    centers = 0.5 * jax.lax.broadcasted_iota(
        jnp.int32, (3, 1), 0).astype(jnp.float32)
    intmax = jnp.int32(2147483647)

    def step(_, carry):
        key = carry[0]
        acc = carry[1:]
        km = jnp.min(key, axis=0, keepdims=True)
        ohb = key == km
        oh = ohb.astype(jnp.float32)
        GkT = _dot(YgT, oh)
        relT = GkT[0:3, :] - TT
        dist = jnp.sqrt(jnp.sum(relT * relT, axis=0, keepdims=True) + 1e-8)
        u = relT / dist
        sh = jnp.concatenate(_sh_cols(u[0:1, :], u[1:2, :], u[2:3, :]),
                             axis=0)
        dn = dist / radius
        g = jnp.exp(-GSCALE * (dn - centers) ** 2)
        g = g * (dn <= 1.0).astype(jnp.float32)
        new_acc = accum_fn(acc, oh, GkT, sh, g)
        key = jnp.where(ohb, intmax, key)
        return (key,) + tuple(new_acc)

    carry = jax.lax.fori_loop(0, PATCH, step, (key0,) + tuple(accum_init))
    return carry[1:]


def _tail(z_list, Nt, u, shb, mw1, mb1, mw2, mb2):
    cat = jnp.concatenate(z_list, axis=1)
    ct = jnp.transpose(cat, (1, 0, 2))
    sig = jax.lax.dot_general(shb, ct, (((1,), (0,)), ((), ())),
                              preferred_element_type=jnp.float32)
    sig = jnp.maximum(sig, 0.0)
    X = sig.reshape(64 * Nt, u)
    X = jnp.maximum(_dot(X, mw1) + mb1, 0.0)
    X = jnp.maximum(_dot(X, mw2) + mb2, 0.0)
    return X


def _make_scale0(Ns, Nt, radius, u, c2):
    def body(T_ref, TT_ref, YT_ref, shb_ref, shbT_ref,
             w0_ref, w1_ref, w2_ref, w3_ref, b0_ref,
             mw1_ref, mb1_ref, mw2_ref, mb2_ref, o_ref):
        T = T_ref[0]
        TT = TT_ref[0]
        YT = YT_ref[0]

        init = tuple(jnp.zeros((16, Nt), jnp.float32) for _ in range(3)) \
            + tuple(jnp.zeros((3, Nt), jnp.float32) for _ in range(3))

        def accum(acc, oh, GkT, sh, g):
            yzx_s = jnp.concatenate(
                [GkT[1:2, :], GkT[2:3, :], GkT[0:1, :]], axis=0)
            out = []
            for s in range(3):
                out.append(acc[s] + sh * g[s:s + 1, :])
            for s in range(3):
                out.append(acc[3 + s] + yzx_s * g[s:s + 1, :])
            return tuple(out)

        acc = _knn_geom(TT, YT, Ns=Ns, Nt=Nt, radius=radius,
                        accum_init=init, accum_fn=accum)
        tAs = [jnp.transpose(a) for a in acc[0:3]]
        tB1s = [jnp.transpose(a) for a in acc[3:6]]
        inv = jnp.float32(1.0 / PATCH)
        yzx_t = jnp.concatenate([T[:, 1:2], T[:, 2:3], T[:, 0:1]],
                                axis=1)[:, :, None]

        def apart(a, b):
            return jnp.concatenate(
                [tAs[s][:, a:b, None] for s in range(3)], axis=-1)

        t0 = apart(0, 1)
        out0 = jnp.concatenate([t0, t0], -1) * inv
        tB1 = jnp.concatenate([tB1s[s][:, :, None] for s in range(3)], -1)
        out1 = jnp.concatenate(
            [apart(1, 4) * inv, tB1 * inv, yzx_t], -1)
        out2 = apart(4, 9) * inv
        out3 = apart(9, 16) * inv
        ws = [w0_ref[...], w1_ref[...], w2_ref[...], w3_ref[...]]
        dims = [(1, 6), (3, 7), (5, 3), (7, 3)]
        z_list = []
        for l, (o_l, (m, din)) in enumerate(zip([out0, out1, out2, out3],
                                                dims)):
            z = _dot(o_l.reshape(Nt * m, din), ws[l])
            if l == 0:
                z = z + b0_ref[...]
            z_list.append(z.reshape(Nt, m, u))
        X = _tail(z_list, Nt, u, shb_ref[...],
                  mw1_ref[...], mb1_ref[...], mw2_ref[...], mb2_ref[...])
        X3 = X.reshape(64, Nt, c2)
        YS = jax.lax.dot_general(shbT_ref[...], X3, (((1,), (0,)), ((), ())),
                                 preferred_element_type=jnp.float32)
        o_ref[0] = YS
    return body


def _make_scale(Ns, Nt, radius, c_in, u, c2, final):
    c0 = c_in
    cls = [c_in, c_in + 1, c_in, c_in]
    offs = []
    off = 3 + c0
    for l in range(4):
        m = 2 * l + 1
        if l == 0:
            offs.append(3)
        else:
            offs.append(off)
            off += m * cls[l]

    def body(*refs):
        (T_ref, TT_ref, YT_ref, shb_ref, shbT_ref,
         w0_ref, w1_ref, w2_ref, w3_ref, b0_ref,
         mw1_ref, mb1_ref, mw2_ref, mb2_ref) = refs[:14]
        if final:
            hw1_ref, hb1_ref, hw2_ref, hb2_ref, o_ref = refs[14:]
        else:
            o_ref = refs[14]
        T = T_ref[0]
        TT = TT_ref[0]
        YT = YT_ref[0]
        YgT = YT[0:3 + c0, :]
        YLT = YT[3:, :]

        init = (jnp.zeros((16, 3 * c0, Nt), jnp.float32),) \
            + tuple(jnp.zeros((Ns, Nt), jnp.float32) for _ in range(3))

        def accum(acc, oh, GkT, sh, g):
            g0 = GkT[3:, :]
            g0cat = jnp.concatenate(
                [g0 * g[0:1, :], g0 * g[1:2, :], g0 * g[2:3, :]],
                axis=0)
            out = [acc[0] + sh[:, None, :] * g0cat[None, :, :]]
            for s in range(3):
                out.append(acc[1 + s] + oh * g[s:s + 1, :])
            return tuple(out)

        acc = _knn_geom(TT, YgT, Ns=Ns, Nt=Nt, radius=radius,
                        accum_init=init, accum_fn=accum)
        tAfull = jnp.transpose(
            acc[0].reshape(16 * 3 * c0, Nt)).reshape(Nt, 16, 3 * c0)
        ows = acc[1:4]
        tBall = [_dot(YLT, ow) for ow in ows]
        inv = jnp.float32(1.0 / PATCH)
        yzx_t = jnp.concatenate([T[:, 1:2], T[:, 2:3], T[:, 0:1]],
                                axis=1)[:, :, None]
        ws = [w0_ref[...], w1_ref[...], w2_ref[...], w3_ref[...]]
        z_list = []
        for l in range(4):
            a, b = L_SLICES[l]
            m = 2 * l + 1
            c = cls[l]
            o = offs[l] - 3
            tA_l = tAfull[:, a:b, :]
            tB_l = jnp.concatenate(
                [jnp.transpose(tBall[s][o:o + m * c, :]).reshape(Nt, m, c)
                 for s in range(3)], axis=-1)
            parts = [tA_l * inv, tB_l * inv]
            if l == 1:
                parts.append(yzx_t)
            o_l = jnp.concatenate(parts, axis=-1)
            din = o_l.shape[-1]
            z = _dot(o_l.reshape(Nt * m, din), ws[l])
            if l == 0:
                z = z + b0_ref[...]
            z_list.append(z.reshape(Nt, m, u))
        X = _tail(z_list, Nt, u, shb_ref[...],
                  mw1_ref[...], mb1_ref[...], mw2_ref[...], mb2_ref[...])
        X3 = X.reshape(64, Nt, c2)
        if final:
            gmax = jnp.max(X3, axis=1)
            h = jnp.maximum(_dot(gmax, hw1_ref[...]) + hb1_ref[...], 0.0)
            code = _dot(h, hw2_ref[...]) + hb2_ref[...]
            lat = _dot(shbT_ref[...], code) * jnp.float32(1.0 / 64.0)
            o_ref[0] = lat
        else:
            YS = jax.lax.dot_general(shbT_ref[...], X3,
                                     (((1,), (0,)), ((), ())),
                                     preferred_element_type=jnp.float32)
            o_ref[0] = YS
    return body


def _bspec(shape, batched):
    if batched:
        blk = (1,) + shape[1:]
        nd = len(shape) - 1
        return pl.BlockSpec(blk, lambda b, _nd=nd: (b,) + (0,) * _nd)
    return pl.BlockSpec(shape, lambda b, _nd=len(shape): (0,) * _nd)


def _call(body, outs_shape, args_batched):
    B = outs_shape[0]
    in_specs = [_bspec(a.shape, bt) for a, bt in args_batched]
    out_spec = _bspec(outs_shape, True)
    return pl.pallas_call(
        body,
        grid=(B,),
        in_specs=in_specs,
        out_specs=out_spec,
        out_shape=jax.ShapeDtypeStruct(outs_shape, jnp.float32),
    )(*[a for a, _ in args_batched])


def kernel(x, params):
    B = x.shape[0]
    p = params
    shb = jnp.asarray(_shb_np())
    mscale = np.zeros((16,), np.float32)
    for l in range(4):
        a, b = L_SLICES[l]
        mscale[a:b] = (2 * l + 1) / 64.0
    shbT_s = jnp.asarray(_shb_np().T * mscale[:, None])
    shbT_f = jnp.asarray(_shb_np().T)

    kc = x[..., 0:1]
    kr = x[..., 0][:, None, :]
    srt = pl.pallas_call(
        _sort_body,
        grid=(B,),
        in_specs=[_bspec(kc.shape, True), _bspec(kr.shape, True),
                  _bspec(x.shape, True)],
        out_specs=_bspec(x.shape, True),
        out_shape=jax.ShapeDtypeStruct(x.shape, jnp.float32),
    )(kc, kr, x)

    points = [srt]
    for i in range(3):
        pts = points[-1]
        points.append(pts.reshape(B, NUM_POINTS[i + 1],
                                  NUM_POINTS[i] // NUM_POINTS[i + 1],
                                  3)[:, :, 0, :])

    def mlp_args(i):
        out = []
        for j in range(2):
            out.append((p['mlp%d_%d_W' % (i, j)], False))
            out.append((p['mlp%d_%d_b' % (i, j)][None, :], False))
        return out

    def eq_args(i):
        out = [(p['eq%d_%d' % (i, l)], False) for l in range(4)]
        out.append((p['eqb%d' % i][None, :], False))
        return out

    def tr(a):
        return jnp.swapaxes(a, 1, 2)

    Nt0 = NUM_POINTS[1]
    body0 = _make_scale0(NUM_POINTS[0], Nt0, RADIUS[0], u=32, c2=32)
    args0 = ([(points[1], True), (tr(points[1]), True),
              (tr(points[0]), True), (shb, False), (shbT_s, False)]
             + eq_args(0) + mlp_args(0))
    YS0 = _call(body0, (B, 16, Nt0, 32), args0)

    def build_Y(pts, YS, c):
        Nt = pts.shape[1]
        parts = [pts]
        for l in range(4):
            a, b = L_SLICES[l]
            yl = jnp.transpose(YS[:, a:b], (0, 2, 1, 3))
            if l == 1:
                yzx = jnp.stack([pts[..., 1], pts[..., 2], pts[..., 0]],
                                axis=-1)[..., None]
                yl = jnp.concatenate([yl, yzx], axis=-1)
            parts.append(yl.reshape(B, Nt, -1))
        return jnp.concatenate(parts, axis=-1)

    Y1 = build_Y(points[1], YS0, 32)
    Nt1 = NUM_POINTS[2]
    body1 = _make_scale(NUM_POINTS[1], Nt1, RADIUS[1], c_in=32, u=64,
                        c2=64, final=False)
    args1 = ([(points[2], True), (tr(points[2]), True),
              (tr(Y1), True), (shb, False), (shbT_s, False)]
             + eq_args(1) + mlp_args(1))
    YS1 = _call(body1, (B, 16, Nt1, 64), args1)

    Y2 = build_Y(points[2], YS1, 64)
    Nt2 = NUM_POINTS[3]
    body2 = _make_scale(NUM_POINTS[2], Nt2, RADIUS[2], c_in=64, u=128,
                        c2=256, final=True)
    args2 = ([(points[3], True), (tr(points[3]), True),
              (tr(Y2), True), (shb, False), (shbT_f, False)]
             + eq_args(2) + mlp_args(2)
             + [(p['code_mlp_W'], False), (p['code_mlp_b'][None, :], False),
                (p['code_W'], False), (p['code_b'][None, :], False)])
    latent = _call(body2, (B, 16, 128), args2)
    return latent


# speedup vs baseline: 194.3175x; 1.0011x over previous
import numpy as np
import jax
import jax.numpy as jnp
from jax.experimental import pallas as pl
from jax.experimental.pallas import tpu as pltpu

L_SLICES = [(0, 1), (1, 4), (4, 9), (9, 16)]
NUM_POINTS = [1024, 256, 64, 16]
RADIUS = [0.2, 0.4, 0.8]
PATCH = 32
NUM_SHELLS = 3
GSCALE = 0.69314718056 * 9
BIG = 1e30


def _shb_np():
    n = 64
    idx = np.arange(0, n, dtype=np.float64) + 0.5
    phi = np.arccos(1 - 2 * idx / n)
    theta = np.pi * (1 + 5 ** 0.5) * idx
    x = (np.cos(theta) * np.sin(phi)).astype(np.float32)
    y = (np.sin(theta) * np.sin(phi)).astype(np.float32)
    z = np.cos(phi).astype(np.float32)
    one = np.ones_like(x)
    comps = [one,
             y, z, x,
             x * y, y * z, 3 * z * z - 1, z * x, x * x - y * y,
             y * (3 * x * x - y * y), x * y * z, y * (5 * z * z - 1),
             z * (5 * z * z - 3),
             x * (5 * z * z - 1), z * (x * x - y * y), x * (x * x - 3 * y * y)]
    return np.stack(comps, axis=-1).astype(np.float32)


def _sh_cols(ux, uy, uz):
    one = jnp.ones_like(ux)
    return [one,
            uy, uz, ux,
            ux * uy, uy * uz, 3 * uz * uz - 1, uz * ux, ux * ux - uy * uy,
            uy * (3 * ux * ux - uy * uy), ux * uy * uz,
            uy * (5 * uz * uz - 1), uz * (5 * uz * uz - 3),
            ux * (5 * uz * uz - 1), uz * (ux * ux - uy * uy),
            ux * (ux * ux - 3 * uy * uy)]


def _sort_body(kc_ref, kr_ref, x_ref, o_ref):
    kc = kc_ref[0]
    kr = kr_ref[0]
    xv = x_ref[0]
    n = kc.shape[0]
    less = (kr < kc).astype(jnp.float32)
    eq = kr == kc
    ii = jax.lax.broadcasted_iota(jnp.int32, (n, n), 0)
    jj = jax.lax.broadcasted_iota(jnp.int32, (n, n), 1)
    eqlt = jnp.where(jnp.logical_and(eq, jj < ii), 1.0, 0.0)
    rank = jnp.sum(less + eqlt, axis=1, keepdims=True)
    rr = jax.lax.broadcasted_iota(jnp.int32, (n, n), 1)
    perm = (rank.astype(jnp.int32) == rr).astype(jnp.float32)
    srt = jax.lax.dot_general(perm, xv, (((0,), (0,)), ((), ())),
                              preferred_element_type=jnp.float32)
    o_ref[0] = srt


def _dot(a, b):
    return jnp.dot(a, b, preferred_element_type=jnp.float32)


def _knn_geom(TT, YgT, Ns, Nt, radius, accum_init, accum_fn):
    S_T = YgT[0:3, :]
    ST = jax.lax.dot_general(S_T, TT, (((0,), (0,)), ((), ())),
                             preferred_element_type=jnp.float32)
    s2col = jnp.transpose(jnp.sum(S_T * S_T, axis=0, keepdims=True))
    t2row = jnp.sum(TT * TT, axis=0, keepdims=True)
    d2 = jnp.clip(s2col - 2.0 * ST + t2row, 0.0, 1.99)
    iota_i = jax.lax.broadcasted_iota(jnp.int32, (Ns, Nt), 0)
    key0 = (d2 * 1048576.0).astype(jnp.int32) * 1024 + iota_i
    centers = 0.5 * jax.lax.broadcasted_iota(
        jnp.int32, (3, 1), 0).astype(jnp.float32)
    intmax = jnp.int32(2147483647)

    def step(_, carry):
        key = carry[0]
        acc = carry[1:]
        km = jnp.min(key, axis=0, keepdims=True)
        ohb = key == km
        oh = ohb.astype(jnp.float32)
        GkT = _dot(YgT, oh)
        relT = GkT[0:3, :] - TT
        dist = jnp.sqrt(jnp.sum(relT * relT, axis=0, keepdims=True) + 1e-8)
        u = relT / dist
        sh = jnp.concatenate(_sh_cols(u[0:1, :], u[1:2, :], u[2:3, :]),
                             axis=0)
        dn = dist / radius
        g = jnp.exp(-GSCALE * (dn - centers) ** 2)
        g = g * (dn <= 1.0).astype(jnp.float32)
        new_acc = accum_fn(acc, oh, GkT, sh, g)
        key = jnp.where(ohb, intmax, key)
        return (key,) + tuple(new_acc)

    carry = jax.lax.fori_loop(0, PATCH, step, (key0,) + tuple(accum_init))
    return carry[1:]


def _tail(z_list, Nt, u, shb, mw1, mb1, mw2, mb2):
    cat = jnp.concatenate(z_list, axis=1)
    ct = jnp.transpose(cat, (1, 0, 2))
    sig = jax.lax.dot_general(shb, ct, (((1,), (0,)), ((), ())),
                              preferred_element_type=jnp.float32)
    sig = jnp.maximum(sig, 0.0)
    X = sig.reshape(64 * Nt, u)
    X = jnp.maximum(_dot(X, mw1) + mb1, 0.0)
    X = jnp.maximum(_dot(X, mw2) + mb2, 0.0)
    return X


def _make_scale0(Ns, Nt, radius, u, c2):
    def body(T_ref, TT_ref, YT_ref, shb_ref, shbT_ref,
             w0_ref, w1_ref, w2_ref, w3_ref, b0_ref,
             mw1_ref, mb1_ref, mw2_ref, mb2_ref, o_ref):
        T = T_ref[0]
        TT = TT_ref[0]
        YT = YT_ref[0]

        init = tuple(jnp.zeros((16, Nt), jnp.float32) for _ in range(3)) \
            + tuple(jnp.zeros((3, Nt), jnp.float32) for _ in range(3))

        def accum(acc, oh, GkT, sh, g):
            yzx_s = jnp.concatenate(
                [GkT[1:2, :], GkT[2:3, :], GkT[0:1, :]], axis=0)
            out = []
            for s in range(3):
                out.append(acc[s] + sh * g[s:s + 1, :])
            for s in range(3):
                out.append(acc[3 + s] + yzx_s * g[s:s + 1, :])
            return tuple(out)

        acc = _knn_geom(TT, YT, Ns=Ns, Nt=Nt, radius=radius,
                        accum_init=init, accum_fn=accum)
        tAs = [jnp.transpose(a) for a in acc[0:3]]
        tB1s = [jnp.transpose(a) for a in acc[3:6]]
        inv = jnp.float32(1.0 / PATCH)
        yzx_t = jnp.concatenate([T[:, 1:2], T[:, 2:3], T[:, 0:1]],
                                axis=1)[:, :, None]

        def apart(a, b):
            return jnp.concatenate(
                [tAs[s][:, a:b, None] for s in range(3)], axis=-1)

        t0 = apart(0, 1)
        out0 = jnp.concatenate([t0, t0], -1) * inv
        tB1 = jnp.concatenate([tB1s[s][:, :, None] for s in range(3)], -1)
        out1 = jnp.concatenate(
            [apart(1, 4) * inv, tB1 * inv, yzx_t], -1)
        out2 = apart(4, 9) * inv
        out3 = apart(9, 16) * inv
        ws = [w0_ref[...], w1_ref[...], w2_ref[...], w3_ref[...]]
        dims = [(1, 6), (3, 7), (5, 3), (7, 3)]
        z_list = []
        for l, (o_l, (m, din)) in enumerate(zip([out0, out1, out2, out3],
                                                dims)):
            z = _dot(o_l.reshape(Nt * m, din), ws[l])
            if l == 0:
                z = z + b0_ref[...]
            z_list.append(z.reshape(Nt, m, u))
        X = _tail(z_list, Nt, u, shb_ref[...],
                  mw1_ref[...], mb1_ref[...], mw2_ref[...], mb2_ref[...])
        X3 = X.reshape(64, Nt, c2)
        YS = jax.lax.dot_general(shbT_ref[...], X3, (((1,), (0,)), ((), ())),
                                 preferred_element_type=jnp.float32)
        o_ref[0] = YS
    return body


def _make_scale(Ns, Nt, radius, c_in, u, c2, final):
    c0 = c_in
    cls = [c_in, c_in + 1, c_in, c_in]
    offs = []
    off = 3 + c0
    for l in range(4):
        m = 2 * l + 1
        if l == 0:
            offs.append(3)
        else:
            offs.append(off)
            off += m * cls[l]

    def body(*refs):
        (T_ref, TT_ref, YT_ref, shb_ref, shbT_ref,
         w0_ref, w1_ref, w2_ref, w3_ref, b0_ref,
         mw1_ref, mb1_ref, mw2_ref, mb2_ref) = refs[:14]
        if final:
            hw1_ref, hb1_ref, hw2_ref, hb2_ref, o_ref = refs[14:]
        else:
            o_ref = refs[14]
        T = T_ref[0]
        TT = TT_ref[0]
        YT = YT_ref[0]
        YgT = YT[0:3 + c0, :]
        YLT = YT[3:, :]

        init = (jnp.zeros((16, 3 * c0, Nt), jnp.float32),) \
            + tuple(jnp.zeros((Ns, Nt), jnp.float32) for _ in range(3))

        def accum(acc, oh, GkT, sh, g):
            g0 = GkT[3:, :]
            g0cat = jnp.concatenate(
                [g0 * g[0:1, :], g0 * g[1:2, :], g0 * g[2:3, :]],
                axis=0)
            out = [acc[0] + sh[:, None, :] * g0cat[None, :, :]]
            for s in range(3):
                out.append(acc[1 + s] + oh * g[s:s + 1, :])
            return tuple(out)

        acc = _knn_geom(TT, YgT, Ns=Ns, Nt=Nt, radius=radius,
                        accum_init=init, accum_fn=accum)
        tAfull = jnp.transpose(
            acc[0].reshape(16 * 3 * c0, Nt)).reshape(Nt, 16, 3 * c0)
        ows = acc[1:4]
        tBall = [_dot(YLT, ow) for ow in ows]
        inv = jnp.float32(1.0 / PATCH)
        yzx_t = jnp.concatenate([T[:, 1:2], T[:, 2:3], T[:, 0:1]],
                                axis=1)[:, :, None]
        ws = [w0_ref[...], w1_ref[...], w2_ref[...], w3_ref[...]]
        z_list = []
        for l in range(4):
            a, b = L_SLICES[l]
            m = 2 * l + 1
            c = cls[l]
            o = offs[l] - 3
            tA_l = tAfull[:, a:b, :]
            tB_l = jnp.concatenate(
                [jnp.transpose(tBall[s][o:o + m * c, :]).reshape(Nt, m, c)
                 for s in range(3)], axis=-1)
            parts = [tA_l * inv, tB_l * inv]
            if l == 1:
                parts.append(yzx_t)
            o_l = jnp.concatenate(parts, axis=-1)
            din = o_l.shape[-1]
            z = _dot(o_l.reshape(Nt * m, din), ws[l])
            if l == 0:
                z = z + b0_ref[...]
            z_list.append(z.reshape(Nt, m, u))
        X = _tail(z_list, Nt, u, shb_ref[...],
                  mw1_ref[...], mb1_ref[...], mw2_ref[...], mb2_ref[...])
        X3 = X.reshape(64, Nt, c2)
        if final:
            gmax = jnp.max(X3, axis=1)
            h = jnp.maximum(_dot(gmax, hw1_ref[...]) + hb1_ref[...], 0.0)
            code = _dot(h, hw2_ref[...]) + hb2_ref[...]
            lat = _dot(shbT_ref[...], code) * jnp.float32(1.0 / 64.0)
            o_ref[0] = lat
        else:
            YS = jax.lax.dot_general(shbT_ref[...], X3,
                                     (((1,), (0,)), ((), ())),
                                     preferred_element_type=jnp.float32)
            o_ref[0] = YS
    return body


def _bspec(shape, batched):
    if batched:
        blk = (1,) + shape[1:]
        nd = len(shape) - 1
        return pl.BlockSpec(blk, lambda b, _nd=nd: (b,) + (0,) * _nd)
    return pl.BlockSpec(shape, lambda b, _nd=len(shape): (0,) * _nd)


def _call(body, outs_shape, args_batched):
    B = outs_shape[0]
    in_specs = [_bspec(a.shape, bt) for a, bt in args_batched]
    out_spec = _bspec(outs_shape, True)
    return pl.pallas_call(
        body,
        grid=(B,),
        in_specs=in_specs,
        out_specs=out_spec,
        out_shape=jax.ShapeDtypeStruct(outs_shape, jnp.float32),
        compiler_params=pltpu.CompilerParams(
            dimension_semantics=("parallel",)),
    )(*[a for a, _ in args_batched])


def kernel(x, params):
    B = x.shape[0]
    p = params
    shb = jnp.asarray(_shb_np())
    mscale = np.zeros((16,), np.float32)
    for l in range(4):
        a, b = L_SLICES[l]
        mscale[a:b] = (2 * l + 1) / 64.0
    shbT_s = jnp.asarray(_shb_np().T * mscale[:, None])
    shbT_f = jnp.asarray(_shb_np().T)

    kc = x[..., 0:1]
    kr = x[..., 0][:, None, :]
    srt = pl.pallas_call(
        _sort_body,
        grid=(B,),
        in_specs=[_bspec(kc.shape, True), _bspec(kr.shape, True),
                  _bspec(x.shape, True)],
        out_specs=_bspec(x.shape, True),
        out_shape=jax.ShapeDtypeStruct(x.shape, jnp.float32),
    )(kc, kr, x)

    points = [srt]
    for i in range(3):
        pts = points[-1]
        points.append(pts.reshape(B, NUM_POINTS[i + 1],
                                  NUM_POINTS[i] // NUM_POINTS[i + 1],
                                  3)[:, :, 0, :])

    def mlp_args(i):
        out = []
        for j in range(2):
            out.append((p['mlp%d_%d_W' % (i, j)], False))
            out.append((p['mlp%d_%d_b' % (i, j)][None, :], False))
        return out

    def eq_args(i):
        out = [(p['eq%d_%d' % (i, l)], False) for l in range(4)]
        out.append((p['eqb%d' % i][None, :], False))
        return out

    def tr(a):
        return jnp.swapaxes(a, 1, 2)

    Nt0 = NUM_POINTS[1]
    body0 = _make_scale0(NUM_POINTS[0], Nt0, RADIUS[0], u=32, c2=32)
    args0 = ([(points[1], True), (tr(points[1]), True),
              (tr(points[0]), True), (shb, False), (shbT_s, False)]
             + eq_args(0) + mlp_args(0))
    YS0 = _call(body0, (B, 16, Nt0, 32), args0)

    def build_Y(pts, YS, c):
        Nt = pts.shape[1]
        parts = [pts]
        for l in range(4):
            a, b = L_SLICES[l]
            yl = jnp.transpose(YS[:, a:b], (0, 2, 1, 3))
            if l == 1:
                yzx = jnp.stack([pts[..., 1], pts[..., 2], pts[..., 0]],
                                axis=-1)[..., None]
                yl = jnp.concatenate([yl, yzx], axis=-1)
            parts.append(yl.reshape(B, Nt, -1))
        return jnp.concatenate(parts, axis=-1)

    Y1 = build_Y(points[1], YS0, 32)
    Nt1 = NUM_POINTS[2]
    body1 = _make_scale(NUM_POINTS[1], Nt1, RADIUS[1], c_in=32, u=64,
                        c2=64, final=False)
    args1 = ([(points[2], True), (tr(points[2]), True),
              (tr(Y1), True), (shb, False), (shbT_s, False)]
             + eq_args(1) + mlp_args(1))
    YS1 = _call(body1, (B, 16, Nt1, 64), args1)

    Y2 = build_Y(points[2], YS1, 64)
    Nt2 = NUM_POINTS[3]
    body2 = _make_scale(NUM_POINTS[2], Nt2, RADIUS[2], c_in=64, u=128,
                        c2=256, final=True)
    args2 = ([(points[3], True), (tr(points[3]), True),
              (tr(Y2), True), (shb, False), (shbT_f, False)]
             + eq_args(2) + mlp_args(2)
             + [(p['code_mlp_W'], False), (p['code_mlp_b'][None, :], False),
                (p['code_W'], False), (p['code_b'][None, :], False)])
    latent = _call(body2, (B, 16, 128), args2)
    return latent
